# Initial kernel scaffold; baseline (speedup 1.0000x reference)
import jax
import jax.numpy as jnp
from jax.experimental import pallas as pl


def kernel(users, items, edge_index, edge_weight, user_emb, item_emb, W_gc_0, b_gc_0, W_bi_0, b_bi_0, W_gc_1, b_gc_1, W_bi_1, b_bi_1, W_gc_2, b_gc_2, W_bi_2, b_bi_2):
    raise NotImplementedError("write your pallas kernel here")


# speedup vs baseline: 1.0005x; 1.0005x over previous
import jax
import jax.numpy as jnp
from jax.experimental import pallas as pl

_NUM_USERS = 30000
_N_LAYERS = 3


def _rating_kernel(u_ref, i_ref, o_ref):
    x = jnp.sum(u_ref[...] * i_ref[...], axis=1)
    o_ref[...] = 1.0 / (1.0 + jnp.exp(-x))


def kernel(users, items, edge_index, edge_weight, user_emb, item_emb, W_gc_0, b_gc_0, W_bi_0, b_bi_0, W_gc_1, b_gc_1, W_bi_1, b_bi_1, W_gc_2, b_gc_2, W_bi_2, b_bi_2):
    Wgc = [W_gc_0, W_gc_1, W_gc_2]
    bgc = [b_gc_0, b_gc_1, b_gc_2]
    Wbi = [W_bi_0, W_bi_1, W_bi_2]
    bbi = [b_bi_0, b_bi_1, b_bi_2]
    src = edge_index[0]
    dst = edge_index[1]
    N = user_emb.shape[0] + item_emb.shape[0]
    ego = jnp.concatenate([user_emb, item_emb], axis=0)
    all_emb = [ego]
    for k in range(_N_LAYERS):
        side = jax.ops.segment_sum(edge_weight[:, None] * ego[src], dst, num_segments=N)
        sum_emb = side @ Wgc[k] + bgc[k]
        bi_emb = (ego * side) @ Wbi[k] + bbi[k]
        ego = jax.nn.leaky_relu(sum_emb + bi_emb, negative_slope=0.2)
        nrm = jnp.maximum(jnp.linalg.norm(ego, axis=1, keepdims=True), 1e-12)
        all_emb.append(ego / nrm)
    all_emb = jnp.concatenate(all_emb, axis=1)
    users_emb = all_emb[users]
    items_emb = all_emb[_NUM_USERS + items]
    return pl.pallas_call(
        _rating_kernel,
        out_shape=jax.ShapeDtypeStruct((users.shape[0],), jnp.float32),
    )(users_emb, items_emb)


# speedup vs baseline: 2.2311x; 2.2299x over previous
import dataclasses
import functools

import jax
import jax.numpy as jnp
from jax import lax
from jax.experimental import pallas as pl
from jax.experimental.pallas import tpu as pltpu
from jax.experimental.pallas import tpu_sc as plsc

_NUM_USERS = 30000
_N = 100000
_DQ = 16
_NQ = 4
_NSUB = 16
_BATCH = 128
_NBATCH = 16
_NCHUNK = 49
_EP_TILE = _NCHUNK * _NBATCH * _BATCH
_EPAD = _EP_TILE * _NSUB
_ROWS_TILE = _EP_TILE // _BATCH

_WCHUNK = 1000
_NWCHUNK = _N // _WCHUNK
_WROUNDS = -(-_NWCHUNK // _NSUB)

_B = 4096
_GB = _B // _BATCH

_VMESH = plsc.VectorSubcoreMesh(core_axis_name="c", subcore_axis_name="s")

_SC_PARAMS = pltpu.CompilerParams()
for _f, _v in (("needs_layout_passes", False), ("use_tc_tiling_on_sc", False)):
    if _f in pltpu.CompilerParams.__dataclass_fields__:
        _SC_PARAMS = dataclasses.replace(_SC_PARAMS, **{_f: _v})


def _sc_segsum_body(ego_hbm, src_hbm, dst_hbm, w_hbm, side_hbm,
                    sidx, didx, wbuf, gbuf, zbuf, acc):
    core = lax.axis_index("c")
    sub = lax.axis_index("s")
    row_base = sub * _ROWS_TILE

    @pl.loop(0, _WCHUNK)
    def _(i):
        zbuf[i] = jnp.zeros((_DQ,), jnp.float32)

    def one_pass(p):
        q = core * 2 + p
        for z in range(_WROUNDS):
            ck = sub + z * _NSUB

            @pl.when(ck < _NWCHUNK)
            def _():
                pltpu.sync_copy(zbuf, acc.at[pl.ds(ck * _WCHUNK, _WCHUNK), :])
        plsc.subcore_barrier()

        @pl.loop(0, _NCHUNK)
        def _(c):
            row0 = row_base + c * _NBATCH
            pltpu.sync_copy(src_hbm.at[pl.ds(row0, _NBATCH)], sidx)
            pltpu.sync_copy(dst_hbm.at[pl.ds(row0, _NBATCH)], didx)
            pltpu.sync_copy(w_hbm.at[pl.ds(row0, _NBATCH)], wbuf)
            for b in range(_NBATCH):
                pltpu.sync_copy(ego_hbm.at[q].at[sidx.at[b]], gbuf)

                @pl.loop(0, _BATCH, step=4)
                def _(j):
                    for u in range(4):
                        wv = plsc.load_gather(
                            wbuf,
                            [jnp.full((_DQ,), b, jnp.int32),
                             jnp.full((_DQ,), j + u, jnp.int32)])
                        gbuf[j + u] = gbuf[j + u] * wv

                pltpu.sync_copy(gbuf, acc.at[didx.at[b]], add=True)
        plsc.subcore_barrier()
        for z in range(_WROUNDS):
            ck = sub + z * _NSUB

            @pl.when(ck < _NWCHUNK)
            def _():
                r0 = ck * _WCHUNK
                pltpu.sync_copy(acc.at[pl.ds(r0, _WCHUNK), :],
                                side_hbm.at[q].at[pl.ds(r0, _WCHUNK)])
        plsc.subcore_barrier()

    one_pass(0)
    one_pass(1)


@jax.jit
def _sc_segsum(egoq, src2, dst2, w2):
    k = pl.kernel(
        _sc_segsum_body,
        out_type=jax.ShapeDtypeStruct((_NQ, _N, _DQ), jnp.float32),
        mesh=_VMESH,
        compiler_params=_SC_PARAMS,
        scratch_types=[
            pltpu.VMEM((_NBATCH, _BATCH), jnp.int32),
            pltpu.VMEM((_NBATCH, _BATCH), jnp.int32),
            pltpu.VMEM((_NBATCH, _BATCH), jnp.float32),
            pltpu.VMEM((_BATCH, _DQ), jnp.float32),
            pltpu.VMEM((_WCHUNK, _DQ), jnp.float32),
            pltpu.VMEM_SHARED((_N, _DQ), jnp.float32),
        ],
    )
    return k(egoq, src2, dst2, w2)


_BLK = 1000


def _dense_body(side_ref, ego_ref, wgc_ref, bgc_ref, wbi_ref, bbi_ref,
                ego_out, nrm_out):
    s = jnp.moveaxis(side_ref[...], 0, 1).reshape(_BLK, 64)
    e = jnp.moveaxis(ego_ref[...], 0, 1).reshape(_BLK, 64)
    h = (jnp.dot(s, wgc_ref[...], preferred_element_type=jnp.float32)
         + bgc_ref[...]
         + jnp.dot(e * s, wbi_ref[...], preferred_element_type=jnp.float32)
         + bbi_ref[...])
    h = jnp.where(h >= 0, h, 0.2 * h)
    nrm = jnp.maximum(jnp.sqrt(jnp.sum(h * h, axis=1, keepdims=True)), 1e-12)
    hn = h / nrm
    ego_out[...] = jnp.moveaxis(h.reshape(_BLK, _NQ, _DQ), 1, 0)
    nrm_out[...] = jnp.moveaxis(hn.reshape(_BLK, _NQ, _DQ), 1, 0)


@jax.jit
def _dense(sideq, egoq, wgc, bgc, wbi, bbi):
    io_spec = pl.BlockSpec((_NQ, _BLK, _DQ), lambda i: (0, i, 0))
    w_spec = pl.BlockSpec((64, 64), lambda i: (0, 0))
    b_spec = pl.BlockSpec((1, 64), lambda i: (0, 0))
    return pl.pallas_call(
        _dense_body,
        grid=(_N // _BLK,),
        in_specs=[io_spec, io_spec, w_spec, b_spec, w_spec, b_spec],
        out_specs=[io_spec, io_spec],
        out_shape=[jax.ShapeDtypeStruct((_NQ, _N, _DQ), jnp.float32),
                   jax.ShapeDtypeStruct((_NQ, _N, _DQ), jnp.float32)],
    )(sideq, egoq, wgc, bgc, wbi, bbi)


def _sc_gather_body(e0, n1, n2, n3, gidx_hbm, out_hbm, idxv, rows):
    core = lax.axis_index("c")
    sub = lax.axis_index("s")
    job = core * _NSUB + sub
    arr = sub // _NQ
    q = sub % _NQ
    for ai, tbl in enumerate((e0, n1, n2, n3)):
        @pl.when(arr == ai)
        def _():
            @pl.loop(0, _GB)
            def _(bt):
                pltpu.sync_copy(gidx_hbm.at[core].at[pl.ds(bt * _BATCH, _BATCH)], idxv)
                pltpu.sync_copy(tbl.at[q].at[idxv], rows)
                pltpu.sync_copy(rows, out_hbm.at[job].at[pl.ds(bt * _BATCH, _BATCH)])


@jax.jit
def _sc_gather(e0, n1, n2, n3, gidx):
    k = pl.kernel(
        _sc_gather_body,
        out_type=jax.ShapeDtypeStruct((2 * _NSUB, _B, _DQ), jnp.float32),
        mesh=_VMESH,
        compiler_params=_SC_PARAMS,
        scratch_types=[
            pltpu.VMEM((_BATCH,), jnp.int32),
            pltpu.VMEM((_BATCH, _DQ), jnp.float32),
        ],
    )
    return k(e0, n1, n2, n3, gidx)


_BB = 512


def _rating_body(g_ref, o_ref):
    g = g_ref[...]
    acc = jnp.zeros((_BB,), jnp.float32)
    for aq in range(_NSUB):
        acc = acc + jnp.sum(g[aq] * g[_NSUB + aq], axis=1)
    o_ref[...] = 1.0 / (1.0 + jnp.exp(-acc))


@jax.jit
def _rating(gathered):
    return pl.pallas_call(
        _rating_body,
        grid=(_B // _BB,),
        in_specs=[pl.BlockSpec((2 * _NSUB, _BB, _DQ), lambda i: (0, i, 0))],
        out_specs=pl.BlockSpec((_BB,), lambda i: (i,)),
        out_shape=jax.ShapeDtypeStruct((_B,), jnp.float32),
    )(gathered)


def kernel(users, items, edge_index, edge_weight, user_emb, item_emb, W_gc_0, b_gc_0, W_bi_0, b_bi_0, W_gc_1, b_gc_1, W_bi_1, b_bi_1, W_gc_2, b_gc_2, W_bi_2, b_bi_2):
    Wgc = [W_gc_0, W_gc_1, W_gc_2]
    bgc = [b_gc_0, b_gc_1, b_gc_2]
    Wbi = [W_bi_0, W_bi_1, W_bi_2]
    bbi = [b_bi_0, b_bi_1, b_bi_2]

    e = edge_index.shape[1]
    pad = _EPAD - e
    src = jnp.concatenate([edge_index[0], jnp.zeros((pad,), edge_index.dtype)])
    dst = jnp.concatenate([edge_index[1], jnp.zeros((pad,), edge_index.dtype)])
    w = jnp.concatenate([edge_weight, jnp.zeros((pad,), edge_weight.dtype)])
    src2 = src.reshape(_EPAD // _BATCH, _BATCH).astype(jnp.int32)
    dst2 = dst.reshape(_EPAD // _BATCH, _BATCH).astype(jnp.int32)
    w2 = w.reshape(_EPAD // _BATCH, _BATCH)

    ego0 = jnp.concatenate([user_emb, item_emb], axis=0)
    egoq = jnp.moveaxis(ego0.reshape(_N, _NQ, _DQ), 1, 0)

    e0 = egoq
    nrms = []
    for k in range(3):
        sideq = _sc_segsum(egoq, src2, dst2, w2)
        egoq, nrmq = _dense(sideq, egoq, Wgc[k], bgc[k], Wbi[k], bbi[k])
        nrms.append(nrmq)

    gidx = jnp.stack([users.astype(jnp.int32),
                      items.astype(jnp.int32) + _NUM_USERS])
    gathered = _sc_gather(e0, nrms[0], nrms[1], nrms[2], gidx)
    return _rating(gathered)


# speedup vs baseline: 2.3183x; 1.0391x over previous
import dataclasses
import functools

import jax
import jax.numpy as jnp
from jax import lax
from jax.experimental import pallas as pl
from jax.experimental.pallas import tpu as pltpu
from jax.experimental.pallas import tpu_sc as plsc

_NUM_USERS = 30000
_N = 100000
_DQ = 16
_NQ = 4
_NSUB = 16
_BATCH = 128
_NBATCH = 16
_NCHUNK = 49
_EP_TILE = _NCHUNK * _NBATCH * _BATCH
_EPAD = _EP_TILE * _NSUB
_ROWS_TILE = _EP_TILE // _BATCH

_WCHUNK = 1000
_NWCHUNK = _N // _WCHUNK
_WROUNDS = -(-_NWCHUNK // _NSUB)

_B = 4096
_GB = _B // _BATCH

_VMESH = plsc.VectorSubcoreMesh(core_axis_name="c", subcore_axis_name="s")

_SC_PARAMS = pltpu.CompilerParams()
for _f, _v in (("needs_layout_passes", False), ("use_tc_tiling_on_sc", False)):
    if _f in pltpu.CompilerParams.__dataclass_fields__:
        _SC_PARAMS = dataclasses.replace(_SC_PARAMS, **{_f: _v})


def _sc_segsum_body(ego_hbm, src_hbm, dst_hbm, w_hbm, side_hbm,
                    sidx, didx, wbuf, gbuf, zbuf, acc):
    core = lax.axis_index("c")
    sub = lax.axis_index("s")
    row_base = sub * _ROWS_TILE

    @pl.loop(0, _WCHUNK)
    def _(i):
        zbuf[i] = jnp.zeros((_DQ,), jnp.float32)

    def one_pass(p):
        q = core * 2 + p
        for z in range(_WROUNDS):
            ck = sub + z * _NSUB

            @pl.when(ck < _NWCHUNK)
            def _():
                pltpu.sync_copy(zbuf, acc.at[pl.ds(ck * _WCHUNK, _WCHUNK), :])
        plsc.subcore_barrier()

        @pl.loop(0, _NCHUNK)
        def _(c):
            row0 = row_base + c * _NBATCH
            pltpu.sync_copy(src_hbm.at[pl.ds(row0, _NBATCH)], sidx)
            pltpu.sync_copy(dst_hbm.at[pl.ds(row0, _NBATCH)], didx)
            pltpu.sync_copy(w_hbm.at[pl.ds(row0, _NBATCH)], wbuf)
            for b in range(_NBATCH):
                pltpu.sync_copy(ego_hbm.at[q].at[sidx.at[b]], gbuf)

                @pl.loop(0, _BATCH, step=4)
                def _(j):
                    for u in range(4):
                        wv = plsc.load_gather(
                            wbuf,
                            [jnp.full((_DQ,), b, jnp.int32),
                             jnp.full((_DQ,), j + u, jnp.int32)])
                        gbuf[j + u] = gbuf[j + u] * wv

                pltpu.sync_copy(gbuf, acc.at[didx.at[b]], add=True)
        plsc.subcore_barrier()
        for z in range(_WROUNDS):
            ck = sub + z * _NSUB

            @pl.when(ck < _NWCHUNK)
            def _():
                r0 = ck * _WCHUNK
                pltpu.sync_copy(acc.at[pl.ds(r0, _WCHUNK), :],
                                side_hbm.at[pl.ds(r0, _WCHUNK), pl.ds(16 * q, _DQ)])
        plsc.subcore_barrier()

    one_pass(0)
    one_pass(1)


@jax.jit
def _sc_segsum(egoq, src2, dst2, w2):
    k = pl.kernel(
        _sc_segsum_body,
        out_type=jax.ShapeDtypeStruct((_N, 64), jnp.float32),
        mesh=_VMESH,
        compiler_params=_SC_PARAMS,
        scratch_types=[
            pltpu.VMEM((_NBATCH, _BATCH), jnp.int32),
            pltpu.VMEM((_NBATCH, _BATCH), jnp.int32),
            pltpu.VMEM((_NBATCH, _BATCH), jnp.float32),
            pltpu.VMEM((_BATCH, _DQ), jnp.float32),
            pltpu.VMEM((_WCHUNK, _DQ), jnp.float32),
            pltpu.VMEM_SHARED((_N, _DQ), jnp.float32),
        ],
    )
    return k(egoq, src2, dst2, w2)


_BLK = 2000


def _dense_body(side_ref, ego_ref, wgc_ref, bgc_ref, wbi_ref, bbi_ref,
                ego_out, nrm_out):
    s = side_ref[...]
    e = ego_ref[...]
    h = (jnp.dot(s, wgc_ref[...], preferred_element_type=jnp.float32)
         + bgc_ref[...]
         + jnp.dot(e * s, wbi_ref[...], preferred_element_type=jnp.float32)
         + bbi_ref[...])
    h = jnp.where(h >= 0, h, 0.2 * h)
    nrm = jnp.maximum(jnp.sqrt(jnp.sum(h * h, axis=1, keepdims=True)), 1e-12)
    ego_out[...] = h
    nrm_out[...] = h / nrm


@jax.jit
def _dense(side2, ego2, wgc, bgc, wbi, bbi):
    io_spec = pl.BlockSpec((_BLK, 64), lambda i: (i, 0))
    w_spec = pl.BlockSpec((64, 64), lambda i: (0, 0))
    b_spec = pl.BlockSpec((1, 64), lambda i: (0, 0))
    ego_new, nrm = pl.pallas_call(
        _dense_body,
        grid=(_N // _BLK,),
        in_specs=[io_spec, io_spec, w_spec, b_spec, w_spec, b_spec],
        out_specs=[io_spec, io_spec],
        out_shape=[jax.ShapeDtypeStruct((_N, 64), jnp.float32),
                   jax.ShapeDtypeStruct((_N, 64), jnp.float32)],
    )(side2, ego2, wgc, bgc, wbi, bbi)
    return ego_new, nrm


def _sc_gather_body(e0, n1, n2, n3, gidx_hbm, out_hbm, idxv, rows):
    core = lax.axis_index("c")
    sub = lax.axis_index("s")
    arr = sub // _NQ
    q = sub % _NQ
    for ai, tbl in enumerate((e0, n1, n2, n3)):
        @pl.when(arr == ai)
        def _():
            @pl.loop(0, _GB)
            def _(bt):
                pltpu.sync_copy(gidx_hbm.at[core].at[pl.ds(bt * _BATCH, _BATCH)], idxv)
                pltpu.sync_copy(tbl.at[q].at[idxv], rows)
                pltpu.sync_copy(rows, out_hbm.at[core].at[pl.ds(bt * _BATCH, _BATCH), sub])


@jax.jit
def _sc_gather(e0, n1, n2, n3, gidx):
    k = pl.kernel(
        _sc_gather_body,
        out_type=jax.ShapeDtypeStruct((2, _B, _NSUB, _DQ), jnp.float32),
        mesh=_VMESH,
        compiler_params=_SC_PARAMS,
        scratch_types=[
            pltpu.VMEM((_BATCH,), jnp.int32),
            pltpu.VMEM((_BATCH, _DQ), jnp.float32),
        ],
    )
    return k(e0, n1, n2, n3, gidx)


_BB = 512


def _rating_body(g_ref, o_ref):
    g = g_ref[...]
    acc = jnp.sum(g[0] * g[1], axis=1)
    o_ref[...] = 1.0 / (1.0 + jnp.exp(-acc))


@jax.jit
def _rating(gathered):
    g2 = gathered.reshape(2, _B, _NSUB * _DQ)
    return pl.pallas_call(
        _rating_body,
        grid=(_B // _BB,),
        in_specs=[pl.BlockSpec((2, _BB, _NSUB * _DQ), lambda i: (0, i, 0))],
        out_specs=pl.BlockSpec((_BB,), lambda i: (i,)),
        out_shape=jax.ShapeDtypeStruct((_B,), jnp.float32),
    )(g2)


def kernel(users, items, edge_index, edge_weight, user_emb, item_emb, W_gc_0, b_gc_0, W_bi_0, b_bi_0, W_gc_1, b_gc_1, W_bi_1, b_bi_1, W_gc_2, b_gc_2, W_bi_2, b_bi_2):
    Wgc = [W_gc_0, W_gc_1, W_gc_2]
    bgc = [b_gc_0, b_gc_1, b_gc_2]
    Wbi = [W_bi_0, W_bi_1, W_bi_2]
    bbi = [b_bi_0, b_bi_1, b_bi_2]

    e = edge_index.shape[1]
    pad = _EPAD - e
    src = jnp.concatenate([edge_index[0], jnp.zeros((pad,), edge_index.dtype)])
    dst = jnp.concatenate([edge_index[1], jnp.zeros((pad,), edge_index.dtype)])
    w = jnp.concatenate([edge_weight, jnp.zeros((pad,), edge_weight.dtype)])
    src2 = src.reshape(_EPAD // _BATCH, _BATCH).astype(jnp.int32)
    dst2 = dst.reshape(_EPAD // _BATCH, _BATCH).astype(jnp.int32)
    w2 = w.reshape(_EPAD // _BATCH, _BATCH)

    ego0 = jnp.concatenate([user_emb, item_emb], axis=0)
    egoq = jnp.moveaxis(ego0.reshape(_N, _NQ, _DQ), 1, 0)

    e0q = egoq
    ego64 = ego0
    nrmqs = []
    for k in range(3):
        side64 = _sc_segsum(egoq, src2, dst2, w2)
        ego64, nrm64 = _dense(side64, ego64, Wgc[k], bgc[k], Wbi[k], bbi[k])
        egoq = jnp.moveaxis(ego64.reshape(_N, _NQ, _DQ), 1, 0)
        nrmqs.append(jnp.moveaxis(nrm64.reshape(_N, _NQ, _DQ), 1, 0))

    gidx = jnp.stack([users.astype(jnp.int32),
                      items.astype(jnp.int32) + _NUM_USERS])
    gathered = _sc_gather(e0q, nrmqs[0], nrmqs[1], nrmqs[2], gidx)
    return _rating(gathered)


# speedup vs baseline: 6.1927x; 2.6712x over previous
import dataclasses
import functools

import jax
import jax.numpy as jnp
from jax import lax
from jax.experimental import pallas as pl
from jax.experimental.pallas import tpu as pltpu
from jax.experimental.pallas import tpu_sc as plsc

_NUM_USERS = 30000
_N = 100000
_DQ = 16
_NQ = 4
_NSUB = 16
_BATCH = 128
_NBATCH = 8
_NCHUNK = 98
_EP_TILE = _NCHUNK * _NBATCH * _BATCH
_EPAD = _EP_TILE * _NSUB
_ROWS_TILE = _EP_TILE // _BATCH

_WCHUNK = 500
_NWCHUNK = _N // _WCHUNK
_WROUNDS = -(-_NWCHUNK // _NSUB)

_B = 4096
_GB = _B // _BATCH

_VMESH = plsc.VectorSubcoreMesh(core_axis_name="c", subcore_axis_name="s")

_SC_PARAMS = pltpu.CompilerParams()
for _f, _v in (("needs_layout_passes", False), ("use_tc_tiling_on_sc", False)):
    if _f in pltpu.CompilerParams.__dataclass_fields__:
        _SC_PARAMS = dataclasses.replace(_SC_PARAMS, **{_f: _v})


def _sc_segsum_body(ego_hbm, src_hbm, dst_hbm, w_hbm, side_hbm,
                    acc, isem, gsem, ssem):
    pl.run_scoped(
        functools.partial(_sc_segsum_inner, ego_hbm, src_hbm, dst_hbm, w_hbm,
                          side_hbm, acc, isem, gsem, ssem),
        pltpu.VMEM((2, _NBATCH, _BATCH), jnp.int32),
        pltpu.VMEM((2, _NBATCH, _BATCH), jnp.int32),
        pltpu.VMEM((2, _NBATCH, _BATCH), jnp.float32),
        pltpu.VMEM((_NBATCH, _BATCH, _DQ), jnp.float32),
        pltpu.VMEM((_WCHUNK, _DQ), jnp.float32),
    )


def _sc_segsum_inner(ego_hbm, src_hbm, dst_hbm, w_hbm, side_hbm,
                     acc, isem, gsem, ssem, sidx, didx, wbuf, gbig, zbuf):
    core = lax.axis_index("c")
    sub = lax.axis_index("s")
    row_base = sub * _ROWS_TILE

    @pl.loop(0, _WCHUNK)
    def _(i):
        zbuf[i] = jnp.zeros((_DQ,), jnp.float32)

    def issue_idx(c, par):
        row0 = row_base + c * _NBATCH
        pltpu.async_copy(src_hbm.at[pl.ds(row0, _NBATCH)], sidx.at[par], isem)
        pltpu.async_copy(dst_hbm.at[pl.ds(row0, _NBATCH)], didx.at[par], isem)
        pltpu.async_copy(w_hbm.at[pl.ds(row0, _NBATCH)], wbuf.at[par], isem)

    def drain_idx():
        pltpu.make_async_copy(src_hbm.at[pl.ds(0, _NBATCH)], sidx.at[0], isem).wait()
        pltpu.make_async_copy(dst_hbm.at[pl.ds(0, _NBATCH)], didx.at[0], isem).wait()
        pltpu.make_async_copy(w_hbm.at[pl.ds(0, _NBATCH)], wbuf.at[0], isem).wait()

    def one_pass(p):
        q = core * 2 + p
        for z in range(_WROUNDS):
            ck = sub + z * _NSUB

            @pl.when(ck < _NWCHUNK)
            def _():
                pltpu.sync_copy(zbuf, acc.at[pl.ds(ck * _WCHUNK, _WCHUNK), :])
        plsc.subcore_barrier()

        issue_idx(0, 0)

        @pl.loop(0, _NCHUNK)
        def _(c):
            par = lax.rem(c, 2)
            drain_idx()

            @pl.when(c + 1 < _NCHUNK)
            def _():
                issue_idx(c + 1, 1 - par)

            gh = [pltpu.async_copy(ego_hbm.at[q].at[sidx.at[par, b]],
                                   gbig.at[b], gsem)
                  for b in range(_NBATCH)]
            sh = []
            for b in range(_NBATCH):
                gh[b].wait()

                @pl.loop(0, _BATCH, step=4)
                def _(j):
                    for u in range(4):
                        wv = plsc.load_gather(
                            wbuf,
                            [jnp.full((_DQ,), par, jnp.int32),
                             jnp.full((_DQ,), b, jnp.int32),
                             jnp.full((_DQ,), j + u, jnp.int32)])
                        gbig[b, j + u] = gbig[b, j + u] * wv

                sh.append(pltpu.async_copy(gbig.at[b], acc.at[didx.at[par, b]],
                                           ssem, add=True))
            for h in sh:
                h.wait()
        plsc.subcore_barrier()
        for z in range(_WROUNDS):
            ck = sub + z * _NSUB

            @pl.when(ck < _NWCHUNK)
            def _():
                r0 = ck * _WCHUNK
                pltpu.sync_copy(acc.at[pl.ds(r0, _WCHUNK), :],
                                side_hbm.at[pl.ds(r0, _WCHUNK), pl.ds(16 * q, _DQ)])
        plsc.subcore_barrier()

    one_pass(0)
    one_pass(1)


@jax.jit
def _sc_segsum(egoq, src2, dst2, w2):
    k = pl.kernel(
        _sc_segsum_body,
        out_type=jax.ShapeDtypeStruct((_N, 64), jnp.float32),
        mesh=_VMESH,
        compiler_params=_SC_PARAMS,
        scratch_types=[
            pltpu.VMEM_SHARED((_N, _DQ), jnp.float32),
            pltpu.SemaphoreType.DMA,
            pltpu.SemaphoreType.DMA,
            pltpu.SemaphoreType.DMA,
        ],
    )
    return k(egoq, src2, dst2, w2)


_BLK = 2000


def _dense_body(side_ref, ego_ref, wgc_ref, bgc_ref, wbi_ref, bbi_ref,
                ego_out, nrm_out):
    s = side_ref[...]
    e = ego_ref[...]
    h = (jnp.dot(s, wgc_ref[...], preferred_element_type=jnp.float32)
         + bgc_ref[...]
         + jnp.dot(e * s, wbi_ref[...], preferred_element_type=jnp.float32)
         + bbi_ref[...])
    h = jnp.where(h >= 0, h, 0.2 * h)
    nrm = jnp.maximum(jnp.sqrt(jnp.sum(h * h, axis=1, keepdims=True)), 1e-12)
    ego_out[...] = h
    nrm_out[...] = h / nrm


@jax.jit
def _dense(side2, ego2, wgc, bgc, wbi, bbi):
    io_spec = pl.BlockSpec((_BLK, 64), lambda i: (i, 0))
    w_spec = pl.BlockSpec((64, 64), lambda i: (0, 0))
    b_spec = pl.BlockSpec((1, 64), lambda i: (0, 0))
    ego_new, nrm = pl.pallas_call(
        _dense_body,
        grid=(_N // _BLK,),
        in_specs=[io_spec, io_spec, w_spec, b_spec, w_spec, b_spec],
        out_specs=[io_spec, io_spec],
        out_shape=[jax.ShapeDtypeStruct((_N, 64), jnp.float32),
                   jax.ShapeDtypeStruct((_N, 64), jnp.float32)],
    )(side2, ego2, wgc, bgc, wbi, bbi)
    return ego_new, nrm


def _sc_gather_body(e0, n1, n2, n3, gidx_hbm, out_hbm, idxv, rows):
    core = lax.axis_index("c")
    sub = lax.axis_index("s")
    arr = sub // _NQ
    q = sub % _NQ
    for ai, tbl in enumerate((e0, n1, n2, n3)):
        @pl.when(arr == ai)
        def _():
            @pl.loop(0, _GB)
            def _(bt):
                pltpu.sync_copy(gidx_hbm.at[core].at[pl.ds(bt * _BATCH, _BATCH)], idxv)
                pltpu.sync_copy(tbl.at[q].at[idxv], rows)
                pltpu.sync_copy(rows, out_hbm.at[core].at[pl.ds(bt * _BATCH, _BATCH), sub])


@jax.jit
def _sc_gather(e0, n1, n2, n3, gidx):
    k = pl.kernel(
        _sc_gather_body,
        out_type=jax.ShapeDtypeStruct((2, _B, _NSUB, _DQ), jnp.float32),
        mesh=_VMESH,
        compiler_params=_SC_PARAMS,
        scratch_types=[
            pltpu.VMEM((_BATCH,), jnp.int32),
            pltpu.VMEM((_BATCH, _DQ), jnp.float32),
        ],
    )
    return k(e0, n1, n2, n3, gidx)


_BB = 512


def _rating_body(g_ref, o_ref):
    g = g_ref[...]
    acc = jnp.sum(g[0] * g[1], axis=1)
    o_ref[...] = 1.0 / (1.0 + jnp.exp(-acc))


@jax.jit
def _rating(gathered):
    g2 = gathered.reshape(2, _B, _NSUB * _DQ)
    return pl.pallas_call(
        _rating_body,
        grid=(_B // _BB,),
        in_specs=[pl.BlockSpec((2, _BB, _NSUB * _DQ), lambda i: (0, i, 0))],
        out_specs=pl.BlockSpec((_BB,), lambda i: (i,)),
        out_shape=jax.ShapeDtypeStruct((_B,), jnp.float32),
    )(g2)


def kernel(users, items, edge_index, edge_weight, user_emb, item_emb, W_gc_0, b_gc_0, W_bi_0, b_bi_0, W_gc_1, b_gc_1, W_bi_1, b_bi_1, W_gc_2, b_gc_2, W_bi_2, b_bi_2):
    Wgc = [W_gc_0, W_gc_1, W_gc_2]
    bgc = [b_gc_0, b_gc_1, b_gc_2]
    Wbi = [W_bi_0, W_bi_1, W_bi_2]
    bbi = [b_bi_0, b_bi_1, b_bi_2]

    e = edge_index.shape[1]
    pad = _EPAD - e
    src = jnp.concatenate([edge_index[0], jnp.zeros((pad,), edge_index.dtype)])
    dst = jnp.concatenate([edge_index[1], jnp.zeros((pad,), edge_index.dtype)])
    w = jnp.concatenate([edge_weight, jnp.zeros((pad,), edge_weight.dtype)])
    src2 = src.reshape(_EPAD // _BATCH, _BATCH).astype(jnp.int32)
    dst2 = dst.reshape(_EPAD // _BATCH, _BATCH).astype(jnp.int32)
    w2 = w.reshape(_EPAD // _BATCH, _BATCH)

    ego0 = jnp.concatenate([user_emb, item_emb], axis=0)
    egoq = jnp.moveaxis(ego0.reshape(_N, _NQ, _DQ), 1, 0)

    e0q = egoq
    ego64 = ego0
    nrmqs = []
    for k in range(3):
        side64 = _sc_segsum(egoq, src2, dst2, w2)
        ego64, nrm64 = _dense(side64, ego64, Wgc[k], bgc[k], Wbi[k], bbi[k])
        egoq = jnp.moveaxis(ego64.reshape(_N, _NQ, _DQ), 1, 0)
        nrmqs.append(jnp.moveaxis(nrm64.reshape(_N, _NQ, _DQ), 1, 0))

    gidx = jnp.stack([users.astype(jnp.int32),
                      items.astype(jnp.int32) + _NUM_USERS])
    gathered = _sc_gather(e0q, nrmqs[0], nrmqs[1], nrmqs[2], gidx)
    return _rating(gathered)
